# Initial kernel scaffold; baseline (speedup 1.0000x reference)
import jax
import jax.numpy as jnp
from jax.experimental import pallas as pl


def kernel(x_p, x_d, edge_attr_p, edge_attr_d, edge_index_p, x_p_batch, nn0_w1, nn0_b1, nn0_w2, nn0_b2, root0, bias0, nn1_w1, nn1_b1, nn1_w2, nn1_b2, root1, bias1, lin0_w, lin0_b, lin1_w, lin1_b):
    raise NotImplementedError("write your pallas kernel here")


# speedup vs baseline: 1.6019x; 1.6019x over previous
import functools

import jax
import jax.numpy as jnp
from jax import lax
from jax.experimental import pallas as pl
from jax.experimental.pallas import tpu as pltpu
from jax.experimental.pallas import tpu_sc as plsc

N = 10000
E = 160000
F_NODE = 32
F_EDGE = 16
HID = 16
EMB = 32
NGRAPH = 64

NC = 2
NS = 16
NW = NC * NS

GRP = 128
GPW = 40
EPW = GRP * GPW
EPAD = NW * EPW
CHG = 8
CHE = CHG * GRP
NCHUNK = GPW // CHG

NPAD = 10240
RPS = NPAD // NS

def _mesh():
    return plsc.VectorSubcoreMesh(core_axis_name="c", subcore_axis_name="s",
                                  num_cores=NC, num_subcores=NS)


@functools.lru_cache(maxsize=None)
def _sc_gather_kernel():
    @functools.partial(
        pl.kernel,
        out_type=jax.ShapeDtypeStruct((EPAD, F_NODE), jnp.float32),
        mesh=_mesh(),
        scratch_types=[
            pltpu.VMEM((CHG, GRP), jnp.int32),
            pltpu.VMEM((CHE, F_NODE), jnp.float32),
            pltpu.SemaphoreType.DMA,
        ],
        compiler_params=pltpu.CompilerParams(use_tc_tiling_on_sc=False),
    )
    def body(table_hbm, idx_hbm, out_hbm, idx_v, rows_v, sem):
        wid = lax.axis_index("s") * NC + lax.axis_index("c")
        for c in range(NCHUNK):
            gbase = wid * GPW + c * CHG
            ebase = wid * EPW + c * CHE
            pltpu.sync_copy(idx_hbm.at[pl.ds(gbase, CHG)], idx_v)
            for g in range(CHG):
                pltpu.async_copy(table_hbm.at[idx_v.at[g]],
                                 rows_v.at[pl.ds(g * GRP, GRP)], sem)
            for g in range(CHG):
                pltpu.make_async_copy(table_hbm.at[idx_v.at[g]],
                                      rows_v.at[pl.ds(g * GRP, GRP)],
                                      sem).wait()
            pltpu.sync_copy(rows_v, out_hbm.at[pl.ds(ebase, CHE)])

    return body


def _sc_gather(table, idx2d):
    return _sc_gather_kernel()(table, idx2d)


@functools.lru_cache(maxsize=None)
def _sc_scatter_kernel():
    @functools.partial(
        pl.kernel,
        out_type=jax.ShapeDtypeStruct((NC, NPAD, F_NODE), jnp.float32),
        mesh=_mesh(),
        scratch_types=[
            pltpu.VMEM((CHG, GRP), jnp.int32),
            pltpu.VMEM((CHE, F_NODE), jnp.float32),
            pltpu.VMEM_SHARED((NPAD, F_NODE), jnp.float32),
        ],
        compiler_params=pltpu.CompilerParams(use_tc_tiling_on_sc=False),
    )
    def body(msg_hbm, idx_hbm, zeros_hbm, out_hbm, idx_v, msg_v, acc_sh):
        cid = lax.axis_index("c")
        sid = lax.axis_index("s")
        wid = sid * NC + cid
        pltpu.sync_copy(zeros_hbm.at[pl.ds(sid * RPS, RPS)],
                        acc_sh.at[pl.ds(sid * RPS, RPS)])
        plsc.subcore_barrier()
        for c in range(NCHUNK):
            gbase = wid * GPW + c * CHG
            ebase = wid * EPW + c * CHE
            pltpu.sync_copy(idx_hbm.at[pl.ds(gbase, CHG)], idx_v)
            pltpu.sync_copy(msg_hbm.at[pl.ds(ebase, CHE)], msg_v)
            for g in range(CHG):
                pltpu.sync_copy(msg_v.at[pl.ds(g * GRP, GRP)],
                                acc_sh.at[idx_v.at[g]], add=True)
        plsc.subcore_barrier()
        pltpu.sync_copy(acc_sh.at[pl.ds(sid * RPS, RPS)],
                        out_hbm.at[cid, pl.ds(sid * RPS, RPS)])

    return body


def _sc_scatter(msg, idx2d, zeros_init):
    return _sc_scatter_kernel()(msg, idx2d, zeros_init)


_MTILE = 2048


def _msg_body(ea_ref, xs_ref, w1_ref, b1_ref, w2r_ref, b2m_ref, out_ref):
    h = jnp.maximum(
        jnp.dot(ea_ref[...], w1_ref[...], preferred_element_type=jnp.float32)
        + b1_ref[...], 0.0)
    xs = xs_ref[...]
    t = jnp.concatenate([xs * h[:, k:k + 1] for k in range(HID)], axis=1)
    out_ref[...] = (
        jnp.dot(t, w2r_ref[...], preferred_element_type=jnp.float32)
        + jnp.dot(xs, b2m_ref[...], preferred_element_type=jnp.float32))


def _msg(ea, xs, w1, b1, w2r, b2m):
    cin = xs.shape[1]
    grid = EPAD // _MTILE
    return pl.pallas_call(
        _msg_body,
        grid=(grid,),
        in_specs=[
            pl.BlockSpec((_MTILE, F_EDGE), lambda i: (i, 0)),
            pl.BlockSpec((_MTILE, cin), lambda i: (i, 0)),
            pl.BlockSpec((F_EDGE, HID), lambda i: (0, 0)),
            pl.BlockSpec((1, HID), lambda i: (0, 0)),
            pl.BlockSpec((HID * cin, EMB), lambda i: (0, 0)),
            pl.BlockSpec((cin, EMB), lambda i: (0, 0)),
        ],
        out_specs=pl.BlockSpec((_MTILE, EMB), lambda i: (i, 0)),
        out_shape=jax.ShapeDtypeStruct((EPAD, EMB), jnp.float32),
    )(ea, xs, w1, b1, w2r, b2m)


def _combine_body(p_ref, x_ref, root_ref, bias_ref, out_ref):
    agg = p_ref[0, :N, :] + p_ref[1, :N, :]
    out_ref[...] = jnp.maximum(
        agg + jnp.dot(x_ref[...], root_ref[...],
                      preferred_element_type=jnp.float32) + bias_ref[...],
        0.0)


def _combine(parts, x, root, bias):
    return pl.pallas_call(
        _combine_body,
        out_shape=jax.ShapeDtypeStruct((N, EMB), jnp.float32),
    )(parts, x, root, bias)


def _final_body(p_ref, x_ref, root_ref, bias_ref, batch_ref,
                l0w_ref, l0b_ref, l1w_ref, l1b_ref, out_ref, gm_ref):
    agg = p_ref[0, :N, :] + p_ref[1, :N, :]
    x2 = jnp.maximum(
        agg + jnp.dot(x_ref[...], root_ref[...],
                      preferred_element_type=jnp.float32) + bias_ref[...],
        0.0)
    batch = batch_ref[...]

    def body(g, _):
        m = jnp.max(jnp.where(batch == g, x2, -jnp.inf), axis=0,
                    keepdims=True)
        gm_ref[pl.ds(g, 1), :] = m
        return 0

    lax.fori_loop(0, NGRAPH, body, 0)
    gm = gm_ref[...]
    y = jnp.dot(gm, l0w_ref[...],
                preferred_element_type=jnp.float32) + l0b_ref[...]
    out_ref[...] = jnp.dot(y, l1w_ref[...],
                           preferred_element_type=jnp.float32) + l1b_ref[...]


def _final(parts, x, root, bias, batch2d, l0w, l0b, l1w, l1b):
    return pl.pallas_call(
        _final_body,
        out_shape=jax.ShapeDtypeStruct((NGRAPH, 1), jnp.float32),
        scratch_shapes=[pltpu.VMEM((NGRAPH, EMB), jnp.float32)],
    )(parts, x, root, bias, batch2d, l0w, l0b, l1w, l1b)


def kernel(x_p, x_d, edge_attr_p, edge_attr_d, edge_index_p, x_p_batch,
           nn0_w1, nn0_b1, nn0_w2, nn0_b2, root0, bias0,
           nn1_w1, nn1_b1, nn1_w2, nn1_b2, root1, bias1,
           lin0_w, lin0_b, lin1_w, lin1_b):
    src = edge_index_p[0]
    dst = edge_index_p[1]
    pad = EPAD - E
    src_p = jnp.concatenate([src, jnp.zeros((pad,), jnp.int32)]
                            ).reshape(EPAD // GRP, GRP)
    dst_p = jnp.concatenate([dst, jnp.full((pad,), N, jnp.int32)]
                            ).reshape(EPAD // GRP, GRP)
    ea_p = jnp.concatenate(
        [edge_attr_p, jnp.zeros((pad, F_EDGE), jnp.float32)], axis=0)
    zeros_init = jnp.zeros((NPAD, F_NODE), jnp.float32)
    batch2d = x_p_batch.reshape(N, 1)

    w2r0 = nn0_w2.reshape(HID, F_NODE, EMB).reshape(HID * F_NODE, EMB)
    b2m0 = nn0_b2.reshape(F_NODE, EMB)
    w2r1 = nn1_w2.reshape(HID, EMB, EMB).reshape(HID * EMB, EMB)
    b2m1 = nn1_b2.reshape(EMB, EMB)

    xs0 = _sc_gather(x_p, src_p)
    msg0 = _msg(ea_p, xs0, nn0_w1, nn0_b1.reshape(1, HID), w2r0, b2m0)
    parts0 = _sc_scatter(msg0, dst_p, zeros_init)
    x1 = _combine(parts0, x_p, root0, bias0.reshape(1, EMB))

    xs1 = _sc_gather(x1, src_p)
    msg1 = _msg(ea_p, xs1, nn1_w1, nn1_b1.reshape(1, HID), w2r1, b2m1)
    parts1 = _sc_scatter(msg1, dst_p, zeros_init)

    return _final(parts1, x1, root1, bias1.reshape(1, EMB), batch2d,
                  lin0_w, lin0_b.reshape(1, EMB), lin1_w,
                  lin1_b.reshape(1, 1))


# speedup vs baseline: 3.0104x; 1.8792x over previous
import functools

import jax
import jax.numpy as jnp
from jax import lax
from jax.experimental import pallas as pl
from jax.experimental.pallas import tpu as pltpu
from jax.experimental.pallas import tpu_sc as plsc

N = 10000
E = 160000
F_NODE = 32
F_EDGE = 16
HID = 16
EMB = 32
NGRAPH = 64

NC = 2
NS = 16
NW = NC * NS

GRP = 128
GPW = 40
EPW = GRP * GPW
EPAD = NW * EPW
CHG = 8
CHE = CHG * GRP
NCHUNK = GPW // CHG

NPAD = 10240
RPS = NPAD // NS

def _mesh():
    return plsc.VectorSubcoreMesh(core_axis_name="c", subcore_axis_name="s",
                                  num_cores=NC, num_subcores=NS)


@functools.lru_cache(maxsize=None)
def _sc_gather_kernel():
    @functools.partial(
        pl.kernel,
        out_type=jax.ShapeDtypeStruct((EPAD, F_NODE), jnp.float32),
        mesh=_mesh(),
        scratch_types=[
            pltpu.VMEM((CHG, GRP), jnp.int32),
            pltpu.VMEM((CHE, F_NODE), jnp.float32),
            pltpu.SemaphoreType.DMA,
        ],
        compiler_params=pltpu.CompilerParams(use_tc_tiling_on_sc=False),
    )
    def body(table_hbm, idx_hbm, out_hbm, idx_v, rows_v, sem):
        wid = lax.axis_index("s") * NC + lax.axis_index("c")
        for c in range(NCHUNK):
            gbase = wid * GPW + c * CHG
            ebase = wid * EPW + c * CHE
            pltpu.sync_copy(idx_hbm.at[pl.ds(gbase, CHG)], idx_v)
            for g in range(CHG):
                pltpu.async_copy(table_hbm.at[idx_v.at[g]],
                                 rows_v.at[pl.ds(g * GRP, GRP)], sem)
            for g in range(CHG):
                pltpu.make_async_copy(table_hbm.at[idx_v.at[g]],
                                      rows_v.at[pl.ds(g * GRP, GRP)],
                                      sem).wait()
            pltpu.sync_copy(rows_v, out_hbm.at[pl.ds(ebase, CHE)])

    return body


def _sc_gather(table, idx2d):
    return _sc_gather_kernel()(table, idx2d)


@functools.lru_cache(maxsize=None)
def _sc_scatter_kernel():
    @functools.partial(
        pl.kernel,
        out_type=jax.ShapeDtypeStruct((NC, NPAD, F_NODE), jnp.float32),
        mesh=_mesh(),
        scratch_types=[
            pltpu.VMEM((CHG, GRP), jnp.int32),
            pltpu.VMEM((CHE, F_NODE), jnp.float32),
            pltpu.VMEM_SHARED((NPAD, F_NODE), jnp.float32),
        ],
        compiler_params=pltpu.CompilerParams(use_tc_tiling_on_sc=False),
    )
    def body(msg_hbm, idx_hbm, zeros_hbm, out_hbm, idx_v, msg_v, acc_sh):
        cid = lax.axis_index("c")
        sid = lax.axis_index("s")
        wid = sid * NC + cid
        pltpu.sync_copy(zeros_hbm.at[pl.ds(sid * RPS, RPS)],
                        acc_sh.at[pl.ds(sid * RPS, RPS)])
        plsc.subcore_barrier()
        for c in range(NCHUNK):
            gbase = wid * GPW + c * CHG
            ebase = wid * EPW + c * CHE
            pltpu.sync_copy(idx_hbm.at[pl.ds(gbase, CHG)], idx_v)
            pltpu.sync_copy(msg_hbm.at[pl.ds(ebase, CHE)], msg_v)
            for g in range(CHG):
                pltpu.sync_copy(msg_v.at[pl.ds(g * GRP, GRP)],
                                acc_sh.at[idx_v.at[g]], add=True)
        plsc.subcore_barrier()
        pltpu.sync_copy(acc_sh.at[pl.ds(sid * RPS, RPS)],
                        out_hbm.at[cid, pl.ds(sid * RPS, RPS)])

    return body


def _sc_scatter(msg, idx2d, zeros_init):
    return _sc_scatter_kernel()(msg, idx2d, zeros_init)


_MTILE = 2048


def _msg_body(ea_ref, xs_ref, w1_ref, b1_ref, w2r_ref, b2m_ref,
              rmat_ref, qmat_ref, out_ref):
    h = jnp.maximum(
        jnp.dot(ea_ref[...], w1_ref[...], preferred_element_type=jnp.float32)
        + b1_ref[...], 0.0)
    xs = xs_ref[...]
    hb = jnp.dot(h, rmat_ref[...], preferred_element_type=jnp.float32)
    xb = jnp.dot(xs, qmat_ref[...], preferred_element_type=jnp.float32)
    out_ref[...] = (
        jnp.dot(hb * xb, w2r_ref[...], preferred_element_type=jnp.float32)
        + jnp.dot(xs, b2m_ref[...], preferred_element_type=jnp.float32))


def _msg(ea, xs, w1, b1, w2r, b2m):
    cin = xs.shape[1]
    kc = HID * cin
    rmat = (jnp.arange(HID)[:, None] == (jnp.arange(kc)[None, :] // cin)
            ).astype(jnp.float32)
    qmat = (jnp.arange(cin)[:, None] == (jnp.arange(kc)[None, :] % cin)
            ).astype(jnp.float32)
    grid = EPAD // _MTILE
    nea = ea.shape[0] // _MTILE + (ea.shape[0] % _MTILE != 0)
    return pl.pallas_call(
        _msg_body,
        grid=(grid,),
        in_specs=[
            pl.BlockSpec((_MTILE, F_EDGE),
                         lambda i: (jnp.minimum(i, nea - 1), 0)),
            pl.BlockSpec((_MTILE, cin), lambda i: (i, 0)),
            pl.BlockSpec((F_EDGE, HID), lambda i: (0, 0)),
            pl.BlockSpec((1, HID), lambda i: (0, 0)),
            pl.BlockSpec((HID * cin, EMB), lambda i: (0, 0)),
            pl.BlockSpec((cin, EMB), lambda i: (0, 0)),
            pl.BlockSpec((HID, kc), lambda i: (0, 0)),
            pl.BlockSpec((cin, kc), lambda i: (0, 0)),
        ],
        out_specs=pl.BlockSpec((_MTILE, EMB), lambda i: (i, 0)),
        out_shape=jax.ShapeDtypeStruct((EPAD, EMB), jnp.float32),
    )(ea, xs, w1, b1, w2r, b2m, rmat, qmat)


def _combine_body(p_ref, x_ref, root_ref, bias_ref, out_ref):
    agg = p_ref[0, :N, :] + p_ref[1, :N, :]
    out_ref[...] = jnp.maximum(
        agg + jnp.dot(x_ref[...], root_ref[...],
                      preferred_element_type=jnp.float32) + bias_ref[...],
        0.0)


def _combine(parts, x, root, bias):
    return pl.pallas_call(
        _combine_body,
        out_shape=jax.ShapeDtypeStruct((N, EMB), jnp.float32),
    )(parts, x, root, bias)


def _final_body(p_ref, x_ref, root_ref, bias_ref, batch_ref,
                l0w_ref, l0b_ref, l1w_ref, l1b_ref, out_ref, gm_ref):
    agg = p_ref[0, :N, :] + p_ref[1, :N, :]
    x2 = jnp.maximum(
        agg + jnp.dot(x_ref[...], root_ref[...],
                      preferred_element_type=jnp.float32) + bias_ref[...],
        0.0)
    batch = batch_ref[...]

    def body(g, _):
        m = jnp.max(jnp.where(batch == g, x2, -jnp.inf), axis=0,
                    keepdims=True)
        gm_ref[pl.ds(g, 1), :] = m
        return 0

    lax.fori_loop(0, NGRAPH, body, 0)
    gm = gm_ref[...]
    y = jnp.dot(gm, l0w_ref[...],
                preferred_element_type=jnp.float32) + l0b_ref[...]
    out_ref[...] = jnp.dot(y, l1w_ref[...],
                           preferred_element_type=jnp.float32) + l1b_ref[...]


def _final(parts, x, root, bias, batch2d, l0w, l0b, l1w, l1b):
    return pl.pallas_call(
        _final_body,
        out_shape=jax.ShapeDtypeStruct((NGRAPH, 1), jnp.float32),
        scratch_shapes=[pltpu.VMEM((NGRAPH, EMB), jnp.float32)],
    )(parts, x, root, bias, batch2d, l0w, l0b, l1w, l1b)


def kernel(x_p, x_d, edge_attr_p, edge_attr_d, edge_index_p, x_p_batch,
           nn0_w1, nn0_b1, nn0_w2, nn0_b2, root0, bias0,
           nn1_w1, nn1_b1, nn1_w2, nn1_b2, root1, bias1,
           lin0_w, lin0_b, lin1_w, lin1_b):
    src = edge_index_p[0]
    dst = edge_index_p[1]
    pad = EPAD - E
    src_p = jnp.concatenate([src, jnp.zeros((pad,), jnp.int32)]
                            ).reshape(EPAD // GRP, GRP)
    dst_p = jnp.concatenate([dst, jnp.full((pad,), N, jnp.int32)]
                            ).reshape(EPAD // GRP, GRP)
    ea_p = edge_attr_p
    zeros_init = jnp.zeros((NPAD, F_NODE), jnp.float32)
    batch2d = x_p_batch.reshape(N, 1)

    w2r0 = nn0_w2.reshape(HID, F_NODE, EMB).reshape(HID * F_NODE, EMB)
    b2m0 = nn0_b2.reshape(F_NODE, EMB)
    w2r1 = nn1_w2.reshape(HID, EMB, EMB).reshape(HID * EMB, EMB)
    b2m1 = nn1_b2.reshape(EMB, EMB)

    xs0 = _sc_gather(x_p, src_p)
    msg0 = _msg(ea_p, xs0, nn0_w1, nn0_b1.reshape(1, HID), w2r0, b2m0)
    parts0 = _sc_scatter(msg0, dst_p, zeros_init)
    x1 = _combine(parts0, x_p, root0, bias0.reshape(1, EMB))

    xs1 = _sc_gather(x1, src_p)
    msg1 = _msg(ea_p, xs1, nn1_w1, nn1_b1.reshape(1, HID), w2r1, b2m1)
    parts1 = _sc_scatter(msg1, dst_p, zeros_init)

    return _final(parts1, x1, root1, bias1.reshape(1, EMB), batch2d,
                  lin0_w, lin0_b.reshape(1, EMB), lin1_w,
                  lin1_b.reshape(1, 1))


# speedup vs baseline: 3.1733x; 1.0541x over previous
import functools

import jax
import jax.numpy as jnp
from jax import lax
from jax.experimental import pallas as pl
from jax.experimental.pallas import tpu as pltpu
from jax.experimental.pallas import tpu_sc as plsc

N = 10000
E = 160000
F_NODE = 32
F_EDGE = 16
HID = 16
EMB = 32
NGRAPH = 64

NC = 2
NS = 16
NW = NC * NS

GRP = 128
GPW = 40
EPW = GRP * GPW
EPAD = NW * EPW
CHG = 8
CHE = CHG * GRP
NCHUNK = GPW // CHG

NPAD = 10240
RPS = NPAD // NS

def _mesh():
    return plsc.VectorSubcoreMesh(core_axis_name="c", subcore_axis_name="s",
                                  num_cores=NC, num_subcores=NS)


@functools.lru_cache(maxsize=None)
def _sc_gather_kernel():
    @functools.partial(
        pl.kernel,
        out_type=jax.ShapeDtypeStruct((EPAD, F_NODE), jnp.float32),
        mesh=_mesh(),
        scratch_types=[
            pltpu.VMEM((CHG, GRP), jnp.int32),
            pltpu.VMEM((CHE, F_NODE), jnp.float32),
            pltpu.SemaphoreType.DMA,
        ],
        compiler_params=pltpu.CompilerParams(use_tc_tiling_on_sc=False),
    )
    def body(table_hbm, idx_hbm, out_hbm, idx_v, rows_v, sem):
        wid = lax.axis_index("s") * NC + lax.axis_index("c")
        for c in range(NCHUNK):
            gbase = wid * GPW + c * CHG
            ebase = wid * EPW + c * CHE
            pltpu.sync_copy(idx_hbm.at[pl.ds(gbase, CHG)], idx_v)
            for g in range(CHG):
                pltpu.async_copy(table_hbm.at[idx_v.at[g]],
                                 rows_v.at[pl.ds(g * GRP, GRP)], sem)
            for g in range(CHG):
                pltpu.make_async_copy(table_hbm.at[idx_v.at[g]],
                                      rows_v.at[pl.ds(g * GRP, GRP)],
                                      sem).wait()
            pltpu.sync_copy(rows_v, out_hbm.at[pl.ds(ebase, CHE)])

    return body


def _sc_gather(table, idx2d):
    return _sc_gather_kernel()(table, idx2d)


@functools.lru_cache(maxsize=None)
def _sc_scatter_kernel():
    @functools.partial(
        pl.kernel,
        out_type=jax.ShapeDtypeStruct((NC, NPAD, F_NODE), jnp.float32),
        mesh=_mesh(),
        scratch_types=[
            pltpu.VMEM((CHG, GRP), jnp.int32),
            pltpu.VMEM((CHE, F_NODE), jnp.float32),
            pltpu.VMEM_SHARED((NPAD, F_NODE), jnp.float32),
        ],
        compiler_params=pltpu.CompilerParams(use_tc_tiling_on_sc=False),
    )
    def body(msg_hbm, idx_hbm, zeros_hbm, out_hbm, idx_v, msg_v, acc_sh):
        cid = lax.axis_index("c")
        sid = lax.axis_index("s")
        wid = sid * NC + cid
        pltpu.sync_copy(zeros_hbm.at[pl.ds(sid * RPS, RPS)],
                        acc_sh.at[pl.ds(sid * RPS, RPS)])
        plsc.subcore_barrier()
        for c in range(NCHUNK):
            gbase = wid * GPW + c * CHG
            ebase = wid * EPW + c * CHE
            pltpu.sync_copy(idx_hbm.at[pl.ds(gbase, CHG)], idx_v)
            pltpu.sync_copy(msg_hbm.at[pl.ds(ebase, CHE)], msg_v)
            for g in range(CHG):
                pltpu.sync_copy(msg_v.at[pl.ds(g * GRP, GRP)],
                                acc_sh.at[idx_v.at[g]], add=True)
        plsc.subcore_barrier()
        pltpu.sync_copy(acc_sh.at[pl.ds(sid * RPS, RPS)],
                        out_hbm.at[cid, pl.ds(sid * RPS, RPS)])

    return body


def _sc_scatter(msg, idx2d, zeros_init):
    return _sc_scatter_kernel()(msg, idx2d, zeros_init)


_MTILE = 2048


def _msg_body(ea_ref, xs_ref, w1e_ref, b1e_ref, vmat_ref, b2m_ref, out_ref):
    hb = jnp.maximum(
        jnp.dot(ea_ref[...], w1e_ref[...], preferred_element_type=jnp.float32)
        + b1e_ref[...], 0.0)
    xs = xs_ref[...]
    p = jnp.dot(xs, vmat_ref[...], preferred_element_type=jnp.float32)
    prod = hb * p
    s = (prod[:, 0:128] + prod[:, 128:256]
         + prod[:, 256:384] + prod[:, 384:512])
    s = s[:, 0:64] + s[:, 64:128]
    s = s[:, 0:32] + s[:, 32:64]
    out_ref[...] = s + jnp.dot(xs, b2m_ref[...],
                               preferred_element_type=jnp.float32)


def _msg(ea, xs, w1, b1, w2, b2):
    cin = xs.shape[1]
    kc = HID * EMB
    w1e = jnp.repeat(w1, EMB, axis=1)
    b1e = jnp.repeat(b1, EMB).reshape(1, kc)
    vmat = (w2.reshape(HID, cin, EMB).transpose(1, 0, 2).reshape(cin, kc))
    b2m = b2.reshape(cin, EMB)
    grid = EPAD // _MTILE
    nea = ea.shape[0] // _MTILE + (ea.shape[0] % _MTILE != 0)
    return pl.pallas_call(
        _msg_body,
        grid=(grid,),
        in_specs=[
            pl.BlockSpec((_MTILE, F_EDGE),
                         lambda i: (jnp.minimum(i, nea - 1), 0)),
            pl.BlockSpec((_MTILE, cin), lambda i: (i, 0)),
            pl.BlockSpec((F_EDGE, kc), lambda i: (0, 0)),
            pl.BlockSpec((1, kc), lambda i: (0, 0)),
            pl.BlockSpec((cin, kc), lambda i: (0, 0)),
            pl.BlockSpec((cin, EMB), lambda i: (0, 0)),
        ],
        out_specs=pl.BlockSpec((_MTILE, EMB), lambda i: (i, 0)),
        out_shape=jax.ShapeDtypeStruct((EPAD, EMB), jnp.float32),
    )(ea, xs, w1e, b1e, vmat, b2m)


def _combine_body(p_ref, x_ref, root_ref, bias_ref, out_ref):
    agg = p_ref[0, :N, :] + p_ref[1, :N, :]
    out_ref[...] = jnp.maximum(
        agg + jnp.dot(x_ref[...], root_ref[...],
                      preferred_element_type=jnp.float32) + bias_ref[...],
        0.0)


def _combine(parts, x, root, bias):
    return pl.pallas_call(
        _combine_body,
        out_shape=jax.ShapeDtypeStruct((N, EMB), jnp.float32),
    )(parts, x, root, bias)


def _final_body(p_ref, x_ref, root_ref, bias_ref, batch_ref,
                l0w_ref, l0b_ref, l1w_ref, l1b_ref, out_ref, gm_ref):
    agg = p_ref[0, :N, :] + p_ref[1, :N, :]
    x2 = jnp.maximum(
        agg + jnp.dot(x_ref[...], root_ref[...],
                      preferred_element_type=jnp.float32) + bias_ref[...],
        0.0)
    batch = batch_ref[...]

    def body(g, _):
        m = jnp.max(jnp.where(batch == g, x2, -jnp.inf), axis=0,
                    keepdims=True)
        gm_ref[pl.ds(g, 1), :] = m
        return 0

    lax.fori_loop(0, NGRAPH, body, 0)
    gm = gm_ref[...]
    y = jnp.dot(gm, l0w_ref[...],
                preferred_element_type=jnp.float32) + l0b_ref[...]
    out_ref[...] = jnp.dot(y, l1w_ref[...],
                           preferred_element_type=jnp.float32) + l1b_ref[...]


def _final(parts, x, root, bias, batch2d, l0w, l0b, l1w, l1b):
    return pl.pallas_call(
        _final_body,
        out_shape=jax.ShapeDtypeStruct((NGRAPH, 1), jnp.float32),
        scratch_shapes=[pltpu.VMEM((NGRAPH, EMB), jnp.float32)],
    )(parts, x, root, bias, batch2d, l0w, l0b, l1w, l1b)


def kernel(x_p, x_d, edge_attr_p, edge_attr_d, edge_index_p, x_p_batch,
           nn0_w1, nn0_b1, nn0_w2, nn0_b2, root0, bias0,
           nn1_w1, nn1_b1, nn1_w2, nn1_b2, root1, bias1,
           lin0_w, lin0_b, lin1_w, lin1_b):
    src = edge_index_p[0]
    dst = edge_index_p[1]
    pad = EPAD - E
    src_p = jnp.concatenate([src, jnp.zeros((pad,), jnp.int32)]
                            ).reshape(EPAD // GRP, GRP)
    dst_p = jnp.concatenate([dst, jnp.full((pad,), N, jnp.int32)]
                            ).reshape(EPAD // GRP, GRP)
    ea_p = edge_attr_p
    zeros_init = jnp.zeros((NPAD, F_NODE), jnp.float32)
    batch2d = x_p_batch.reshape(N, 1)

    xs0 = _sc_gather(x_p, src_p)
    msg0 = _msg(ea_p, xs0, nn0_w1, nn0_b1, nn0_w2, nn0_b2)
    parts0 = _sc_scatter(msg0, dst_p, zeros_init)
    x1 = _combine(parts0, x_p, root0, bias0.reshape(1, EMB))

    xs1 = _sc_gather(x1, src_p)
    msg1 = _msg(ea_p, xs1, nn1_w1, nn1_b1, nn1_w2, nn1_b2)
    parts1 = _sc_scatter(msg1, dst_p, zeros_init)

    return _final(parts1, x1, root1, bias1.reshape(1, EMB), batch2d,
                  lin0_w, lin0_b.reshape(1, EMB), lin1_w,
                  lin1_b.reshape(1, 1))


# speedup vs baseline: 4.7251x; 1.4890x over previous
import functools

import jax
import jax.numpy as jnp
from jax import lax
from jax.experimental import pallas as pl
from jax.experimental.pallas import tpu as pltpu
from jax.experimental.pallas import tpu_sc as plsc

N = 10000
E = 160000
F_NODE = 32
F_EDGE = 16
HID = 16
EMB = 32
NGRAPH = 64

NC = 2
NS = 16
NW = NC * NS

GRP = 128
GPW = 40
EPW = GRP * GPW
EPAD = NW * EPW
CHG = 8
CHE = CHG * GRP
NCHUNK = GPW // CHG

NPAD = 10240
RPS = NPAD // NS

def _mesh():
    return plsc.VectorSubcoreMesh(core_axis_name="c", subcore_axis_name="s",
                                  num_cores=NC, num_subcores=NS)


@functools.lru_cache(maxsize=None)
def _sc_gather_kernel():
    @functools.partial(
        pl.kernel,
        out_type=jax.ShapeDtypeStruct((EPAD, F_NODE), jnp.float32),
        mesh=_mesh(),
        scratch_types=[
            pltpu.VMEM((CHG, GRP), jnp.int32),
            pltpu.VMEM((CHE, F_NODE), jnp.float32),
            pltpu.SemaphoreType.DMA,
        ],
        compiler_params=pltpu.CompilerParams(use_tc_tiling_on_sc=False),
    )
    def body(table_hbm, idx_hbm, out_hbm, idx_v, rows_v, sem):
        out_rows = out_hbm
        wid = lax.axis_index("s") * NC + lax.axis_index("c")
        for c in range(NCHUNK):
            gbase = wid * GPW + c * CHG
            ebase = wid * EPW + c * CHE
            pltpu.sync_copy(idx_hbm.at[pl.ds(gbase, CHG)], idx_v)
            for g in range(CHG):
                pltpu.async_copy(table_hbm.at[idx_v.at[g]],
                                 rows_v.at[pl.ds(g * GRP, GRP)], sem)
            for g in range(CHG):
                pltpu.make_async_copy(table_hbm.at[idx_v.at[g]],
                                      rows_v.at[pl.ds(g * GRP, GRP)],
                                      sem).wait()
            pltpu.sync_copy(rows_v, out_rows.at[pl.ds(ebase, CHE)])

    return body


def _sc_gather(table, idx2d):
    return _sc_gather_kernel()(table, idx2d)


@functools.lru_cache(maxsize=None)
def _sc_scatter_kernel():
    @functools.partial(
        pl.kernel,
        out_type=jax.ShapeDtypeStruct((NC, NPAD, F_NODE), jnp.float32),
        mesh=_mesh(),
        scratch_types=[
            pltpu.VMEM((CHG, GRP), jnp.int32),
            pltpu.VMEM((CHE, F_NODE), jnp.float32),
            pltpu.VMEM_SHARED((NPAD, F_NODE), jnp.float32),
        ],
        compiler_params=pltpu.CompilerParams(use_tc_tiling_on_sc=False),
    )
    def body(msg_hbm, idx_hbm, zeros_hbm, out_hbm, idx_v, msg_v, acc_sh):
        msg_rows = msg_hbm
        cid = lax.axis_index("c")
        sid = lax.axis_index("s")
        wid = sid * NC + cid
        pltpu.sync_copy(zeros_hbm.at[pl.ds(sid * RPS, RPS)],
                        acc_sh.at[pl.ds(sid * RPS, RPS)])
        plsc.subcore_barrier()
        for c in range(NCHUNK):
            gbase = wid * GPW + c * CHG
            ebase = wid * EPW + c * CHE
            pltpu.sync_copy(idx_hbm.at[pl.ds(gbase, CHG)], idx_v)
            pltpu.sync_copy(msg_rows.at[pl.ds(ebase, CHE)], msg_v)
            for g in range(CHG):
                pltpu.sync_copy(msg_v.at[pl.ds(g * GRP, GRP)],
                                acc_sh.at[idx_v.at[g]], add=True)
        plsc.subcore_barrier()
        pltpu.sync_copy(acc_sh.at[pl.ds(sid * RPS, RPS)],
                        out_hbm.at[cid, pl.ds(sid * RPS, RPS)])

    return body


def _sc_scatter(msg, idx2d, zeros_init):
    return _sc_scatter_kernel()(msg, idx2d, zeros_init)


_MTILE = 2048


_BT = _MTILE // 4


def _msg_body(ea_ref, xs_ref, w1e_ref, b1e_ref, vmat_ref, b2m_ref, out_ref):
    hb = jnp.maximum(
        jnp.dot(ea_ref[...], w1e_ref[...], preferred_element_type=jnp.float32)
        + b1e_ref[...], 0.0)
    xs = xs_ref[...]
    p = jnp.dot(xs, vmat_ref[...], preferred_element_type=jnp.float32)
    prod = hb * p
    outs = []
    for j in range(4):
        q = prod[:, 512 * j:512 * (j + 1)]
        s = (q[:, 0:128] + q[:, 128:256] + q[:, 256:384] + q[:, 384:512])
        s = s[:, 0:64] + s[:, 64:128]
        outs.append(s[:, 0:32] + s[:, 32:64])
    out_ref[...] = (jnp.concatenate(outs, axis=1)
                    + jnp.dot(xs, b2m_ref[...],
                              preferred_element_type=jnp.float32))


def _msg(ea4, xs4, w1, b1, w2, b2):
    cin = 32
    kc = HID * EMB
    eye4 = jnp.eye(4, dtype=jnp.float32)
    w1e = jnp.kron(eye4, jnp.repeat(w1, EMB, axis=1))
    b1e = jnp.tile(jnp.repeat(b1, EMB), 4).reshape(1, 4 * kc)
    vmat = jnp.kron(
        eye4,
        w2.reshape(HID, cin, EMB).transpose(1, 0, 2).reshape(cin, kc))
    b2m = jnp.kron(eye4, b2.reshape(cin, EMB))
    grid = (EPAD // 4) // _BT
    nea = ea4.shape[0] // _BT + (ea4.shape[0] % _BT != 0)
    return pl.pallas_call(
        _msg_body,
        grid=(grid,),
        in_specs=[
            pl.BlockSpec((_BT, 4 * F_EDGE),
                         lambda i: (jnp.minimum(i, nea - 1), 0)),
            pl.BlockSpec((_BT, 128), lambda i: (i, 0)),
            pl.BlockSpec((4 * F_EDGE, 4 * kc), lambda i: (0, 0)),
            pl.BlockSpec((1, 4 * kc), lambda i: (0, 0)),
            pl.BlockSpec((128, 4 * kc), lambda i: (0, 0)),
            pl.BlockSpec((128, 128), lambda i: (0, 0)),
        ],
        out_specs=pl.BlockSpec((_BT, 128), lambda i: (i, 0)),
        out_shape=jax.ShapeDtypeStruct((EPAD // 4, 128), jnp.float32),
    )(ea4, xs4, w1e, b1e, vmat, b2m)


def _combine_body(p_ref, x_ref, root_ref, bias_ref, out_ref):
    agg = p_ref[0, :N, :] + p_ref[1, :N, :]
    out_ref[...] = jnp.maximum(
        agg + jnp.dot(x_ref[...], root_ref[...],
                      preferred_element_type=jnp.float32) + bias_ref[...],
        0.0)


def _combine(parts, x, root, bias):
    return pl.pallas_call(
        _combine_body,
        out_shape=jax.ShapeDtypeStruct((N, EMB), jnp.float32),
    )(parts, x, root, bias)


def _final_body(p_ref, x_ref, root_ref, bias_ref, batch_ref,
                l0w_ref, l0b_ref, l1w_ref, l1b_ref, out_ref, gm_ref, x2_ref):
    agg = p_ref[0, :N, :] + p_ref[1, :N, :]
    x2_ref[...] = jnp.maximum(
        agg + jnp.dot(x_ref[...], root_ref[...],
                      preferred_element_type=jnp.float32) + bias_ref[...],
        0.0)
    gm_ref[...] = jnp.full((NGRAPH, EMB), -jnp.inf, jnp.float32)

    ck = 500
    nchunks = N // ck

    def chunk_body(c, _):
        rows = x2_ref[pl.ds(c * ck, ck), :]
        b = batch_ref[pl.ds(c * ck, ck), :]
        glo = b[0, 0]
        ghi = b[ck - 1, 0]

        def g_body(g, _):
            m = jnp.max(jnp.where(b == g, rows, -jnp.inf), axis=0,
                        keepdims=True)
            cur = gm_ref[pl.ds(g, 1), :]
            gm_ref[pl.ds(g, 1), :] = jnp.maximum(cur, m)
            return 0

        lax.fori_loop(glo, ghi + 1, g_body, 0, unroll=False)
        return 0

    lax.fori_loop(0, nchunks, chunk_body, 0, unroll=False)
    gm = gm_ref[...]
    y = jnp.dot(gm, l0w_ref[...],
                preferred_element_type=jnp.float32) + l0b_ref[...]
    out_ref[...] = jnp.dot(y, l1w_ref[...],
                           preferred_element_type=jnp.float32) + l1b_ref[...]


def _final(parts, x, root, bias, batch2d, l0w, l0b, l1w, l1b):
    return pl.pallas_call(
        _final_body,
        out_shape=jax.ShapeDtypeStruct((NGRAPH, 1), jnp.float32),
        scratch_shapes=[pltpu.VMEM((NGRAPH, EMB), jnp.float32),
                        pltpu.VMEM((N, EMB), jnp.float32)],
    )(parts, x, root, bias, batch2d, l0w, l0b, l1w, l1b)


def kernel(x_p, x_d, edge_attr_p, edge_attr_d, edge_index_p, x_p_batch,
           nn0_w1, nn0_b1, nn0_w2, nn0_b2, root0, bias0,
           nn1_w1, nn1_b1, nn1_w2, nn1_b2, root1, bias1,
           lin0_w, lin0_b, lin1_w, lin1_b):
    src = edge_index_p[0]
    dst = edge_index_p[1]
    pad = EPAD - E
    src_p = jnp.concatenate([src, jnp.zeros((pad,), jnp.int32)]
                            ).reshape(EPAD // GRP, GRP)
    dst_p = jnp.concatenate([dst, jnp.full((pad,), N, jnp.int32)]
                            ).reshape(EPAD // GRP, GRP)
    ea4 = edge_attr_p.reshape(E // 4, 4 * F_EDGE)
    zeros_init = jnp.zeros((NPAD, F_NODE), jnp.float32)
    batch2d = x_p_batch.reshape(N, 1)

    xs0 = _sc_gather(x_p, src_p).reshape(EPAD // 4, 128)
    msg0 = _msg(ea4, xs0, nn0_w1, nn0_b1, nn0_w2, nn0_b2)
    parts0 = _sc_scatter(msg0.reshape(EPAD, F_NODE), dst_p, zeros_init)
    x1 = _combine(parts0, x_p, root0, bias0.reshape(1, EMB))

    xs1 = _sc_gather(x1, src_p).reshape(EPAD // 4, 128)
    msg1 = _msg(ea4, xs1, nn1_w1, nn1_b1, nn1_w2, nn1_b2)
    parts1 = _sc_scatter(msg1.reshape(EPAD, F_NODE), dst_p, zeros_init)

    return _final(parts1, x1, root1, bias1.reshape(1, EMB), batch2d,
                  lin0_w, lin0_b.reshape(1, EMB), lin1_w,
                  lin1_b.reshape(1, 1))
